# baseline (speedup 1.0000x reference)
import jax
import jax.numpy as jnp
from jax.experimental import pallas as pl


def _cheb_conv(x, edge_index, W, b):
    src = edge_index[0]
    dst = edge_index[1]
    n = x.shape[0]
    e = src.shape[0]
    deg = jax.ops.segment_sum(jnp.ones((e,), jnp.float32), dst, num_segments=n)
    dinv = jnp.where(deg > 0, 1.0 / jnp.sqrt(jnp.where(deg > 0, deg, 1.0)), 0.0)
    norm = -(dinv[src] * dinv[dst])

    def prop(h):
        return jax.ops.segment_sum(norm[:, None] * h[src], dst, num_segments=n)

    Tx0 = x
    out = Tx0 @ W[0]
    Tx1 = prop(Tx0)
    out = out + Tx1 @ W[1]
    Tx2 = 2.0 * prop(Tx1) - Tx0
    out = out + Tx2 @ W[2]
    return out + b


def _pool(x):
    L = (x.shape[0] + 6) // 4
    return x[:L]


def _hex_upsample(x, up_idx):
    return jnp.concatenate([x, jnp.mean(x[up_idx], axis=1)], axis=0)


def _softmax_pallas(h):
    def body(x_ref, o_ref):
        x = x_ref[...]
        m = jnp.max(x, axis=1, keepdims=True)
        e = jnp.exp(x - m)
        o_ref[...] = e / jnp.sum(e, axis=1, keepdims=True)

    n, f = h.shape
    bn = 2048
    npad = (-n) % bn
    if npad:
        h = jnp.pad(h, ((0, npad), (0, 0)))
    out = pl.pallas_call(
        body,
        grid=((n + npad) // bn,),
        in_specs=[pl.BlockSpec((bn, f), lambda i: (i, 0))],
        out_specs=pl.BlockSpec((bn, f), lambda i: (i, 0)),
        out_shape=jax.ShapeDtypeStruct((n + npad, f), h.dtype),
    )(h)
    return out[:n]


def kernel(x, edge_index, edge_index_1, edge_index_2, edge_index_3, edge_index_4, edge_index_5, up2, up3, up4, up5, up6, W1, b1, W2, b2, W3, b3, W4, b4, W5, b5, W6, b6, W7, b7, W8, b8, W9, b9, W10, b10, W11, b11):
    act = jax.nn.relu
    h = act(_cheb_conv(x, edge_index, W1, b1))
    x1 = _pool(h)
    h = act(_cheb_conv(x1, edge_index_5, W2, b2))
    x2 = _pool(h)
    h = act(_cheb_conv(x2, edge_index_4, W3, b3))
    x3 = _pool(h)
    h = act(_cheb_conv(x3, edge_index_3, W4, b4))
    x4 = _pool(h)
    h = act(_cheb_conv(x4, edge_index_2, W5, b5))
    x5 = _pool(h)
    h = act(_cheb_conv(x5, edge_index_1, W6, b6))
    h = _hex_upsample(h, up2)
    h = jnp.concatenate([h, x4], axis=1)
    h = act(_cheb_conv(h, edge_index_2, W7, b7))
    h = _hex_upsample(h, up3)
    h = jnp.concatenate([h, x3], axis=1)
    h = act(_cheb_conv(h, edge_index_3, W8, b8))
    h = _hex_upsample(h, up4)
    h = jnp.concatenate([h, x2], axis=1)
    h = act(_cheb_conv(h, edge_index_4, W9, b9))
    h = _hex_upsample(h, up5)
    h = jnp.concatenate([h, x1], axis=1)
    h = act(_cheb_conv(h, edge_index_5, W10, b10))
    h = _hex_upsample(h, up6)
    h = jnp.concatenate([h, x], axis=1)
    h = _cheb_conv(h, edge_index, W11, b11)
    return _softmax_pallas(h)


# speedup vs baseline: 5.2048x; 5.2048x over previous
import functools

import jax
import jax.numpy as jnp
from jax import lax
from jax.experimental import pallas as pl
from jax.experimental.pallas import tpu as pltpu
from jax.experimental.pallas import tpu_sc as plsc

NC = 2
NS = 16
NW = NC * NS
EC = 128


def _rup(x, m):
    return -(-x // m) * m


def _zero_fill(ref, nrows, width):
    z = jnp.zeros((16,), jnp.float32)

    def body(i, _):
        for j in range(width // 16):
            ref[i, pl.ds(16 * j, 16)] = z
        if width % 16:
            ref[i, pl.ds(width - 16, 16)] = z
        return 0

    lax.fori_loop(0, nrows, body, 0)


def _copy_slabs(src_at, dst_at, tid, rpt, zr, nzc, nacc):
    for z in range(nzc):
        start = jnp.minimum(tid * rpt + z * zr, nacc - zr)
        pltpu.sync_copy(src_at(start), dst_at(start))


@functools.cache
def _make_deg(N, Ep):
    Nacc = _rup(N + 1, 8)
    e_per_w = Ep // NW
    n_chunks = e_per_w // EC
    rpt = _rup(-(-Nacc // NS), 8)
    zr = min(rpt, 128)
    nzc = -(-rpt // zr)
    mesh = plsc.VectorSubcoreMesh(core_axis_name="c", subcore_axis_name="s")

    @functools.partial(
        pl.kernel,
        mesh=mesh,
        compiler_params=pltpu.CompilerParams(use_tc_tiling_on_sc=False),
        out_type=(
            jax.ShapeDtypeStruct((Nacc, 16), jnp.float32),
            jax.ShapeDtypeStruct((Nacc, 16), jnp.float32),
        ),
        scratch_types=[
            pltpu.VMEM((EC,), jnp.int32),
            pltpu.VMEM((EC, 16), jnp.float32),
            pltpu.VMEM((zr, 16), jnp.float32),
            pltpu.VMEM_SHARED((Nacc, 16), jnp.float32),
        ],
    )
    def deg_kernel(dst_hbm, out0, out1, dst_v, ones_v, zrows_v, acc):
        cid = lax.axis_index("c")
        tid = lax.axis_index("s")
        wid = tid * NC + cid

        one = jnp.ones((16,), jnp.float32)

        def fill_ones(i, _):
            ones_v[i, pl.ds(0, 16)] = one
            return 0

        lax.fori_loop(0, EC, fill_ones, 0)
        _zero_fill(zrows_v, zr, 16)
        _copy_slabs(
            lambda s: zrows_v.at[:],
            lambda s: acc.at[pl.ds(s, zr), :],
            tid, rpt, zr, nzc, Nacc,
        )
        plsc.subcore_barrier()

        def chunk(k, _):
            base = wid * e_per_w + k * EC
            pltpu.sync_copy(dst_hbm.at[pl.ds(base, EC)], dst_v)
            pltpu.sync_copy(ones_v, acc.at[dst_v], add=True)
            return 0

        lax.fori_loop(0, n_chunks, chunk, 0)
        plsc.subcore_barrier()

        @pl.when(cid == 0)
        def _():
            _copy_slabs(
                lambda s: acc.at[pl.ds(s, zr), :],
                lambda s: out0.at[pl.ds(s, zr), :],
                tid, rpt, zr, nzc, Nacc,
            )

        @pl.when(cid == 1)
        def _():
            _copy_slabs(
                lambda s: acc.at[pl.ds(s, zr), :],
                lambda s: out1.at[pl.ds(s, zr), :],
                tid, rpt, zr, nzc, Nacc,
            )

    return deg_kernel


@functools.cache
def _make_segsum(N, Fp, Ep):
    Nacc = _rup(N + 1, 8)
    e_per_w = Ep // NW
    n_chunks = e_per_w // EC
    rpt = _rup(-(-Nacc // NS), 8)
    zr = min(rpt, 128)
    nzc = -(-rpt // zr)
    mesh = plsc.VectorSubcoreMesh(core_axis_name="c", subcore_axis_name="s")

    @functools.partial(
        pl.kernel,
        mesh=mesh,
        compiler_params=pltpu.CompilerParams(use_tc_tiling_on_sc=False),
        out_type=(
            jax.ShapeDtypeStruct((Nacc, Fp), jnp.float32),
            jax.ShapeDtypeStruct((Nacc, Fp), jnp.float32),
        ),
        scratch_types=[
            pltpu.VMEM((EC,), jnp.int32),
            pltpu.VMEM((EC,), jnp.int32),
            pltpu.VMEM((EC, Fp), jnp.float32),
            pltpu.VMEM((zr, Fp), jnp.float32),
            pltpu.VMEM_SHARED((Nacc, Fp), jnp.float32),
            pltpu.SemaphoreType.DMA,
        ],
    )
    def segsum_kernel(g_hbm, src_hbm, dst_hbm, out0, out1,
                      src_v, dst_v, rows_v, zrows_v, acc, sem):
        cid = lax.axis_index("c")
        tid = lax.axis_index("s")
        wid = tid * NC + cid

        _zero_fill(zrows_v, zr, Fp)
        _copy_slabs(
            lambda s: zrows_v.at[:],
            lambda s: acc.at[pl.ds(s, zr), :],
            tid, rpt, zr, nzc, Nacc,
        )
        plsc.subcore_barrier()

        def chunk(k, _):
            base = wid * e_per_w + k * EC
            pltpu.sync_copy(src_hbm.at[pl.ds(base, EC)], src_v)
            pltpu.sync_copy(dst_hbm.at[pl.ds(base, EC)], dst_v)
            pltpu.async_copy(g_hbm.at[src_v], rows_v, sem).wait()
            pltpu.sync_copy(rows_v, acc.at[dst_v], add=True)
            return 0

        lax.fori_loop(0, n_chunks, chunk, 0)
        plsc.subcore_barrier()

        @pl.when(cid == 0)
        def _():
            _copy_slabs(
                lambda s: acc.at[pl.ds(s, zr), :],
                lambda s: out0.at[pl.ds(s, zr), :],
                tid, rpt, zr, nzc, Nacc,
            )

        @pl.when(cid == 1)
        def _():
            _copy_slabs(
                lambda s: acc.at[pl.ds(s, zr), :],
                lambda s: out1.at[pl.ds(s, zr), :],
                tid, rpt, zr, nzc, Nacc,
            )

    return segsum_kernel


@functools.cache
def _make_upsample(Nprev, F, Rp):
    r_per_w = Rp // NW
    cu = min(r_per_w, 128)
    n_chunks = -(-r_per_w // cu)
    mesh = plsc.VectorSubcoreMesh(core_axis_name="c", subcore_axis_name="s")

    @functools.partial(
        pl.kernel,
        mesh=mesh,
        compiler_params=pltpu.CompilerParams(use_tc_tiling_on_sc=False),
        out_type=jax.ShapeDtypeStruct((Rp, F), jnp.float32),
        scratch_types=[
            pltpu.VMEM((cu,), jnp.int32),
            pltpu.VMEM((cu,), jnp.int32),
            pltpu.VMEM((cu, F), jnp.float32),
            pltpu.VMEM((cu, F), jnp.float32),
            pltpu.SemaphoreType.DMA,
        ],
    )
    def up_kernel(h_hbm, i0_hbm, i1_hbm, out, i0_v, i1_v, r0_v, r1_v, sem):
        cid = lax.axis_index("c")
        tid = lax.axis_index("s")
        wid = tid * NC + cid

        for k in range(n_chunks):
            base = jnp.minimum(wid * r_per_w + k * cu, Rp - cu)
            pltpu.sync_copy(i0_hbm.at[pl.ds(base, cu)], i0_v)
            pltpu.sync_copy(i1_hbm.at[pl.ds(base, cu)], i1_v)
            pltpu.async_copy(h_hbm.at[i0_v], r0_v, sem).wait()
            pltpu.async_copy(h_hbm.at[i1_v], r1_v, sem).wait()

            def row(r, _):
                for j in range(F // 16):
                    s = pl.ds(16 * j, 16)
                    r0_v[r, s] = (r0_v[r, s] + r1_v[r, s]) * 0.5
                return 0

            lax.fori_loop(0, cu, row, 0)
            pltpu.sync_copy(r0_v, out.at[pl.ds(base, cu), :])

    return up_kernel


def _dinv_block(d0_ref, d1_ref):
    deg = d0_ref[:, :1] + d1_ref[:, :1]
    return jnp.where(deg > 0, lax.rsqrt(jnp.maximum(deg, 1.0)), 0.0)


@functools.cache
def _make_scale1(N, Fi, Fp, bn):
    def body(h_ref, d0_ref, d1_ref, o_ref):
        dinv = _dinv_block(d0_ref, d1_ref)
        o_ref[:, :Fi] = h_ref[...] * dinv
        if Fp > Fi:
            o_ref[:, Fi:] = jnp.zeros((bn, Fp - Fi), jnp.float32)

    grid = -(-N // bn)
    return pl.pallas_call(
        body,
        grid=(grid,),
        in_specs=[
            pl.BlockSpec((bn, Fi), lambda i: (i, 0)),
            pl.BlockSpec((bn, 16), lambda i: (i, 0)),
            pl.BlockSpec((bn, 16), lambda i: (i, 0)),
        ],
        out_specs=pl.BlockSpec((bn, Fp), lambda i: (i, 0)),
        out_shape=jax.ShapeDtypeStruct((N, Fp), jnp.float32),
    )


@functools.cache
def _make_scale2(N, Fp, bn):
    def body(s1a_ref, s1b_ref, d0_ref, d1_ref, o_ref):
        deg = d0_ref[:, :1] + d1_ref[:, :1]
        idinv2 = jnp.where(deg > 0, -1.0 / jnp.maximum(deg, 1.0), 0.0)
        o_ref[...] = (s1a_ref[...] + s1b_ref[...]) * idinv2

    grid = -(-N // bn)
    return pl.pallas_call(
        body,
        grid=(grid,),
        in_specs=[
            pl.BlockSpec((bn, Fp), lambda i: (i, 0)),
            pl.BlockSpec((bn, Fp), lambda i: (i, 0)),
            pl.BlockSpec((bn, 16), lambda i: (i, 0)),
            pl.BlockSpec((bn, 16), lambda i: (i, 0)),
        ],
        out_specs=pl.BlockSpec((bn, Fp), lambda i: (i, 0)),
        out_shape=jax.ShapeDtypeStruct((N, Fp), jnp.float32),
    )


@functools.cache
def _make_combine(N, Fi, Fp, Fo, bn, final):
    def body(h_ref, s1a_ref, s1b_ref, s2a_ref, s2b_ref, d0_ref, d1_ref,
             w_ref, b_ref, o_ref):
        dinv = _dinv_block(d0_ref, d1_ref)
        s1 = (s1a_ref[:, :Fi] + s1b_ref[:, :Fi]) * dinv
        s2 = (s2a_ref[:, :Fi] + s2b_ref[:, :Fi]) * dinv
        w0 = w_ref[0]
        w1 = w_ref[1]
        w2 = w_ref[2]
        acc = jnp.dot(h_ref[...], w0 - w2, preferred_element_type=jnp.float32)
        acc = acc - jnp.dot(s1, w1, preferred_element_type=jnp.float32)
        acc = acc - 2.0 * jnp.dot(s2, w2, preferred_element_type=jnp.float32)
        acc = acc + b_ref[...]
        if final:
            m = jnp.max(acc, axis=1, keepdims=True)
            e = jnp.exp(acc - m)
            o_ref[...] = e / jnp.sum(e, axis=1, keepdims=True)
        else:
            o_ref[...] = jnp.maximum(acc, 0.0)

    grid = -(-N // bn)
    return pl.pallas_call(
        body,
        grid=(grid,),
        in_specs=[
            pl.BlockSpec((bn, Fi), lambda i: (i, 0)),
            pl.BlockSpec((bn, Fp), lambda i: (i, 0)),
            pl.BlockSpec((bn, Fp), lambda i: (i, 0)),
            pl.BlockSpec((bn, Fp), lambda i: (i, 0)),
            pl.BlockSpec((bn, Fp), lambda i: (i, 0)),
            pl.BlockSpec((bn, 16), lambda i: (i, 0)),
            pl.BlockSpec((bn, 16), lambda i: (i, 0)),
            pl.BlockSpec((3, Fi, Fo), lambda i: (0, 0, 0)),
            pl.BlockSpec((1, Fo), lambda i: (0, 0)),
        ],
        out_specs=pl.BlockSpec((bn, Fo), lambda i: (i, 0)),
        out_shape=jax.ShapeDtypeStruct((N, Fo), jnp.float32),
    )


def _prep_edges(edge_index, N):
    E = edge_index.shape[1]
    Ep = _rup(E, NW * EC)
    src = jnp.pad(edge_index[0], (0, Ep - E))
    dst = jnp.pad(edge_index[1], (0, Ep - E), constant_values=N)
    return src, dst


def _cheb(h, src, dst, deg0, deg1, W, b, final=False):
    N, Fi = h.shape
    Fp = _rup(Fi, 16)
    if _rup(N + 1, 8) * Fp > 1_800_000:
        Fp = _rup(Fi, 8)
    Fo = W.shape[2]
    Ep = src.shape[0]
    bn = min(1024, _rup(N, 8))
    g0 = _make_scale1(N, Fi, Fp, bn)(h, deg0, deg1)
    s1a, s1b = _make_segsum(N, Fp, Ep)(g0, src, dst)
    g1 = _make_scale2(N, Fp, bn)(s1a, s1b, deg0, deg1)
    s2a, s2b = _make_segsum(N, Fp, Ep)(g1, src, dst)
    return _make_combine(N, Fi, Fp, Fo, bn, final)(
        h, s1a, s1b, s2a, s2b, deg0, deg1, W, b.reshape(1, -1))


def _upsample(h, up):
    Nprev, F = h.shape
    R = up.shape[0]
    r_per_w = _rup(-(-R // NW), 8)
    Rp = NW * r_per_w
    i0 = jnp.pad(up[:, 0], (0, Rp - R))
    i1 = jnp.pad(up[:, 1], (0, Rp - R))
    new = _make_upsample(Nprev, F, Rp)(h, i0, i1)
    return jnp.concatenate([h, new[:R]], axis=0)


def kernel(x, edge_index, edge_index_1, edge_index_2, edge_index_3,
           edge_index_4, edge_index_5, up2, up3, up4, up5, up6,
           W1, b1, W2, b2, W3, b3, W4, b4, W5, b5, W6, b6, W7, b7,
           W8, b8, W9, b9, W10, b10, W11, b11):
    N6, N5, N4, N3, N2, N1 = 40962, 10242, 2562, 642, 162, 42
    lv = {}
    for n, ei in ((N6, edge_index), (N5, edge_index_5), (N4, edge_index_4),
                  (N3, edge_index_3), (N2, edge_index_2), (N1, edge_index_1)):
        src, dst = _prep_edges(ei, n)
        d0, d1 = _make_deg(n, src.shape[0])(dst)
        lv[n] = (src, dst, d0, d1)

    h = _cheb(x, *lv[N6], W1, b1)
    x1 = h[:N5]
    h = _cheb(x1, *lv[N5], W2, b2)
    x2 = h[:N4]
    h = _cheb(x2, *lv[N4], W3, b3)
    x3 = h[:N3]
    h = _cheb(x3, *lv[N3], W4, b4)
    x4 = h[:N2]
    h = _cheb(x4, *lv[N2], W5, b5)
    x5 = h[:N1]
    h = _cheb(x5, *lv[N1], W6, b6)

    h = jnp.concatenate([_upsample(h, up2), x4], axis=1)
    h = _cheb(h, *lv[N2], W7, b7)
    h = jnp.concatenate([_upsample(h, up3), x3], axis=1)
    h = _cheb(h, *lv[N3], W8, b8)
    h = jnp.concatenate([_upsample(h, up4), x2], axis=1)
    h = _cheb(h, *lv[N4], W9, b9)
    h = jnp.concatenate([_upsample(h, up5), x1], axis=1)
    h = _cheb(h, *lv[N5], W10, b10)
    h = jnp.concatenate([_upsample(h, up6), x], axis=1)
    return _cheb(h, *lv[N6], W11, b11, final=True)


# speedup vs baseline: 6.0870x; 1.1695x over previous
import functools

import jax
import jax.numpy as jnp
from jax import lax
from jax.experimental import pallas as pl
from jax.experimental.pallas import tpu as pltpu
from jax.experimental.pallas import tpu_sc as plsc

NC = 2
NS = 16
NW = NC * NS
EC = 128


def _rup(x, m):
    return -(-x // m) * m


def _zero_fill(ref, nrows, width):
    z = jnp.zeros((16,), jnp.float32)

    def body(i, _):
        for j in range(width // 16):
            ref[i, pl.ds(16 * j, 16)] = z
        if width % 16:
            ref[i, pl.ds(width - 16, 16)] = z
        return 0

    lax.fori_loop(0, nrows, body, 0)


def _copy_slabs(src_at, dst_at, tid, rpt, zr, nzc, nacc):
    for z in range(nzc):
        start = jnp.minimum(tid * rpt + z * zr, nacc - zr)
        pltpu.sync_copy(src_at(start), dst_at(start))


def _acc_geom(N):
    Nacc = _rup(N + 1, 8)
    rpt = _rup(-(-Nacc // NS), 8)
    zr = min(rpt, 64)
    nzc = -(-rpt // zr)
    return Nacc, rpt, zr, nzc


@functools.cache
def _make_deg(N, Ep):
    Nacc, rpt, zr, nzc = _acc_geom(N)
    n_chunks = Ep // EC // NW
    mesh = plsc.VectorSubcoreMesh(core_axis_name="c", subcore_axis_name="s")

    @functools.partial(
        pl.kernel,
        mesh=mesh,
        compiler_params=pltpu.CompilerParams(use_tc_tiling_on_sc=False),
        out_type=(
            jax.ShapeDtypeStruct((Nacc, 16), jnp.float32),
            jax.ShapeDtypeStruct((Nacc, 16), jnp.float32),
        ),
        scratch_types=[
            pltpu.VMEM((min(n_chunks, 16), EC), jnp.int32),
            pltpu.VMEM((EC, 16), jnp.float32),
            pltpu.VMEM((zr, 16), jnp.float32),
            pltpu.VMEM_SHARED((Nacc, 16), jnp.float32),
        ],
    )
    def deg_kernel(dst_hbm, out0, out1, dst_slab, ones_v, zrows_v, acc):
        cid = lax.axis_index("c")
        tid = lax.axis_index("s")
        wid = tid * NC + cid

        one = jnp.ones((16,), jnp.float32)

        def fill_ones(i, _):
            ones_v[i, pl.ds(0, 16)] = one
            return 0

        lax.fori_loop(0, EC, fill_ones, 0)
        _zero_fill(zrows_v, zr, 16)
        _copy_slabs(
            lambda s: zrows_v.at[:],
            lambda s: acc.at[pl.ds(s, zr), :],
            tid, rpt, zr, nzc, Nacc,
        )
        plsc.subcore_barrier()

        sb = min(n_chunks, 16)
        for s0 in range(0, n_chunks, sb):
            sbc = min(sb, n_chunks - s0)
            pltpu.sync_copy(
                dst_hbm.at[pl.ds(wid * n_chunks + s0, sbc), :],
                dst_slab.at[pl.ds(0, sbc)])
            for k2 in range(sbc):
                pltpu.sync_copy(ones_v, acc.at[dst_slab.at[k2]], add=True)
        plsc.subcore_barrier()

        @pl.when(cid == 0)
        def _():
            _copy_slabs(
                lambda s: acc.at[pl.ds(s, zr), :],
                lambda s: out0.at[pl.ds(s, zr), :],
                tid, rpt, zr, nzc, Nacc,
            )

        @pl.when(cid == 1)
        def _():
            _copy_slabs(
                lambda s: acc.at[pl.ds(s, zr), :],
                lambda s: out1.at[pl.ds(s, zr), :],
                tid, rpt, zr, nzc, Nacc,
            )

    return deg_kernel


@functools.cache
def _make_segsum(N, Fp, Ep):
    Nacc, rpt, zr, nzc = _acc_geom(N)
    n_chunks = Ep // EC // NW
    db = min(3 if Fp <= 192 else 1, n_chunks)
    mesh = plsc.VectorSubcoreMesh(core_axis_name="c", subcore_axis_name="s")

    @functools.partial(
        pl.kernel,
        mesh=mesh,
        compiler_params=pltpu.CompilerParams(use_tc_tiling_on_sc=False),
        out_type=(
            jax.ShapeDtypeStruct((Nacc, Fp), jnp.float32),
            jax.ShapeDtypeStruct((Nacc, Fp), jnp.float32),
        ),
        scratch_types=[
            pltpu.VMEM((min(n_chunks, 16), EC), jnp.int32),
            pltpu.VMEM((min(n_chunks, 16), EC), jnp.int32),
            [pltpu.VMEM((EC, Fp), jnp.float32) for _ in range(db)],
            [pltpu.SemaphoreType.DMA for _ in range(db)],
            pltpu.VMEM((zr, Fp), jnp.float32),
            pltpu.VMEM_SHARED((Nacc, Fp), jnp.float32),
        ],
    )
    def segsum_kernel(g_hbm, src_hbm, dst_hbm, out0, out1,
                      src_slab, dst_slab, rows, sems, zrows_v, acc):
        cid = lax.axis_index("c")
        tid = lax.axis_index("s")
        wid = tid * NC + cid

        _zero_fill(zrows_v, zr, Fp)
        _copy_slabs(
            lambda s: zrows_v.at[:],
            lambda s: acc.at[pl.ds(s, zr), :],
            tid, rpt, zr, nzc, Nacc,
        )
        plsc.subcore_barrier()

        sb = min(n_chunks, 16)
        for s0 in range(0, n_chunks, sb):
            sbc = min(sb, n_chunks - s0)
            pltpu.sync_copy(
                src_hbm.at[pl.ds(wid * n_chunks + s0, sbc), :],
                src_slab.at[pl.ds(0, sbc)])
            pltpu.sync_copy(
                dst_hbm.at[pl.ds(wid * n_chunks + s0, sbc), :],
                dst_slab.at[pl.ds(0, sbc)])
            dbc = min(db, sbc)
            descs = [None] * dbc
            for k in range(dbc):
                descs[k] = pltpu.async_copy(
                    g_hbm.at[src_slab.at[k]], rows[k], sems[k])
            for k in range(sbc):
                b = k % dbc
                descs[b].wait()
                pltpu.sync_copy(rows[b], acc.at[dst_slab.at[k]], add=True)
                nk = k + dbc
                if nk < sbc:
                    descs[b] = pltpu.async_copy(
                        g_hbm.at[src_slab.at[nk]], rows[b], sems[b])
        plsc.subcore_barrier()

        @pl.when(cid == 0)
        def _():
            _copy_slabs(
                lambda s: acc.at[pl.ds(s, zr), :],
                lambda s: out0.at[pl.ds(s, zr), :],
                tid, rpt, zr, nzc, Nacc,
            )

        @pl.when(cid == 1)
        def _():
            _copy_slabs(
                lambda s: acc.at[pl.ds(s, zr), :],
                lambda s: out1.at[pl.ds(s, zr), :],
                tid, rpt, zr, nzc, Nacc,
            )

    return segsum_kernel


@functools.cache
def _make_upsample(Nprev, F, Rp, cu):
    r_per_w = Rp // NW
    n_chunks = r_per_w // cu
    mesh = plsc.VectorSubcoreMesh(core_axis_name="c", subcore_axis_name="s")

    @functools.partial(
        pl.kernel,
        mesh=mesh,
        compiler_params=pltpu.CompilerParams(use_tc_tiling_on_sc=False),
        out_type=jax.ShapeDtypeStruct((Rp, F), jnp.float32),
        scratch_types=[
            pltpu.VMEM((n_chunks, cu), jnp.int32),
            pltpu.VMEM((n_chunks, cu), jnp.int32),
            pltpu.VMEM((cu, F), jnp.float32),
            pltpu.VMEM((cu, F), jnp.float32),
            pltpu.SemaphoreType.DMA,
            pltpu.SemaphoreType.DMA,
        ],
    )
    def up_kernel(h_hbm, i0_hbm, i1_hbm, out, i0_slab, i1_slab,
                  r0_v, r1_v, sem0, sem1):
        cid = lax.axis_index("c")
        tid = lax.axis_index("s")
        wid = tid * NC + cid

        pltpu.sync_copy(i0_hbm.at[pl.ds(wid * n_chunks, n_chunks), :], i0_slab)
        pltpu.sync_copy(i1_hbm.at[pl.ds(wid * n_chunks, n_chunks), :], i1_slab)
        for k in range(n_chunks):
            base = (wid * n_chunks + k) * cu
            d0 = pltpu.async_copy(h_hbm.at[i0_slab.at[k]], r0_v, sem0)
            d1 = pltpu.async_copy(h_hbm.at[i1_slab.at[k]], r1_v, sem1)
            d0.wait()
            d1.wait()

            def row(r, _):
                for j in range(F // 16):
                    s = pl.ds(16 * j, 16)
                    r0_v[r, s] = (r0_v[r, s] + r1_v[r, s]) * 0.5
                return 0

            lax.fori_loop(0, cu, row, 0)
            pltpu.sync_copy(r0_v, out.at[pl.ds(base, cu), :])

    return up_kernel


def _dinv_block(d0_ref, d1_ref):
    deg = d0_ref[:, :1] + d1_ref[:, :1]
    return jnp.where(deg > 0, lax.rsqrt(jnp.maximum(deg, 1.0)), 0.0)


@functools.cache
def _make_scale1(N, Fi, Fp, bn):
    def body(h_ref, d0_ref, d1_ref, o_ref):
        dinv = _dinv_block(d0_ref, d1_ref)
        o_ref[:, :Fi] = h_ref[...] * dinv
        if Fp > Fi:
            o_ref[:, Fi:] = jnp.zeros((bn, Fp - Fi), jnp.float32)

    grid = -(-N // bn)
    return pl.pallas_call(
        body,
        grid=(grid,),
        in_specs=[
            pl.BlockSpec((bn, Fi), lambda i: (i, 0)),
            pl.BlockSpec((bn, 16), lambda i: (i, 0)),
            pl.BlockSpec((bn, 16), lambda i: (i, 0)),
        ],
        out_specs=pl.BlockSpec((bn, Fp), lambda i: (i, 0)),
        out_shape=jax.ShapeDtypeStruct((N, Fp), jnp.float32),
    )


@functools.cache
def _make_scale2(N, Fp, bn):
    def body(s1a_ref, s1b_ref, d0_ref, d1_ref, o_ref):
        deg = d0_ref[:, :1] + d1_ref[:, :1]
        idinv2 = jnp.where(deg > 0, -1.0 / jnp.maximum(deg, 1.0), 0.0)
        o_ref[...] = (s1a_ref[...] + s1b_ref[...]) * idinv2

    grid = -(-N // bn)
    return pl.pallas_call(
        body,
        grid=(grid,),
        in_specs=[
            pl.BlockSpec((bn, Fp), lambda i: (i, 0)),
            pl.BlockSpec((bn, Fp), lambda i: (i, 0)),
            pl.BlockSpec((bn, 16), lambda i: (i, 0)),
            pl.BlockSpec((bn, 16), lambda i: (i, 0)),
        ],
        out_specs=pl.BlockSpec((bn, Fp), lambda i: (i, 0)),
        out_shape=jax.ShapeDtypeStruct((N, Fp), jnp.float32),
    )


@functools.cache
def _make_combine(N, Fi, Fp, Fo, bn, final):
    def body(h_ref, s1a_ref, s1b_ref, s2a_ref, s2b_ref, d0_ref, d1_ref,
             w_ref, b_ref, o_ref):
        dinv = _dinv_block(d0_ref, d1_ref)
        s1 = (s1a_ref[:, :Fi] + s1b_ref[:, :Fi]) * dinv
        s2 = (s2a_ref[:, :Fi] + s2b_ref[:, :Fi]) * dinv
        w0 = w_ref[0]
        w1 = w_ref[1]
        w2 = w_ref[2]
        acc = jnp.dot(h_ref[...], w0 - w2, preferred_element_type=jnp.float32)
        acc = acc - jnp.dot(s1, w1, preferred_element_type=jnp.float32)
        acc = acc - 2.0 * jnp.dot(s2, w2, preferred_element_type=jnp.float32)
        acc = acc + b_ref[...]
        if final:
            m = jnp.max(acc, axis=1, keepdims=True)
            e = jnp.exp(acc - m)
            o_ref[...] = e / jnp.sum(e, axis=1, keepdims=True)
        else:
            o_ref[...] = jnp.maximum(acc, 0.0)

    grid = -(-N // bn)
    return pl.pallas_call(
        body,
        grid=(grid,),
        in_specs=[
            pl.BlockSpec((bn, Fi), lambda i: (i, 0)),
            pl.BlockSpec((bn, Fp), lambda i: (i, 0)),
            pl.BlockSpec((bn, Fp), lambda i: (i, 0)),
            pl.BlockSpec((bn, Fp), lambda i: (i, 0)),
            pl.BlockSpec((bn, Fp), lambda i: (i, 0)),
            pl.BlockSpec((bn, 16), lambda i: (i, 0)),
            pl.BlockSpec((bn, 16), lambda i: (i, 0)),
            pl.BlockSpec((3, Fi, Fo), lambda i: (0, 0, 0)),
            pl.BlockSpec((1, Fo), lambda i: (0, 0)),
        ],
        out_specs=pl.BlockSpec((bn, Fo), lambda i: (i, 0)),
        out_shape=jax.ShapeDtypeStruct((N, Fo), jnp.float32),
    )


def _prep_edges(edge_index, N):
    E = edge_index.shape[1]
    Ep = _rup(E, NW * EC)
    src = jnp.pad(edge_index[0], (0, Ep - E)).reshape(Ep // EC, EC)
    dst = jnp.pad(edge_index[1], (0, Ep - E),
                  constant_values=N).reshape(Ep // EC, EC)
    return src, dst


def _cheb(h, src, dst, deg0, deg1, W, b, final=False):
    N, Fi = h.shape
    Fp = _rup(Fi, 16)
    if _rup(N + 1, 8) * Fp > 1_800_000:
        Fp = _rup(Fi, 8)
    Fo = W.shape[2]
    Ep = src.shape[0] * EC
    bn = min(1024, _rup(N, 8))
    g0 = _make_scale1(N, Fi, Fp, bn)(h, deg0, deg1)
    s1a, s1b = _make_segsum(N, Fp, Ep)(g0, src, dst)
    g1 = _make_scale2(N, Fp, bn)(s1a, s1b, deg0, deg1)
    s2a, s2b = _make_segsum(N, Fp, Ep)(g1, src, dst)
    return _make_combine(N, Fi, Fp, Fo, bn, final)(
        h, s1a, s1b, s2a, s2b, deg0, deg1, W, b.reshape(1, -1))


def _upsample(h, up):
    Nprev, F = h.shape
    R = up.shape[0]
    r_per_w = _rup(-(-R // NW), 8)
    cu = max(d for d in range(8, 129, 8) if r_per_w % d == 0)
    Rp = NW * r_per_w
    i0 = jnp.pad(up[:, 0], (0, Rp - R)).reshape(Rp // cu, cu)
    i1 = jnp.pad(up[:, 1], (0, Rp - R)).reshape(Rp // cu, cu)
    new = _make_upsample(Nprev, F, Rp, cu)(h, i0, i1)
    return jnp.concatenate([h, new[:R]], axis=0)


def kernel(x, edge_index, edge_index_1, edge_index_2, edge_index_3,
           edge_index_4, edge_index_5, up2, up3, up4, up5, up6,
           W1, b1, W2, b2, W3, b3, W4, b4, W5, b5, W6, b6, W7, b7,
           W8, b8, W9, b9, W10, b10, W11, b11):
    N6, N5, N4, N3, N2, N1 = 40962, 10242, 2562, 642, 162, 42
    lv = {}
    for n, ei in ((N6, edge_index), (N5, edge_index_5), (N4, edge_index_4),
                  (N3, edge_index_3), (N2, edge_index_2), (N1, edge_index_1)):
        src, dst = _prep_edges(ei, n)
        d0, d1 = _make_deg(n, src.shape[0] * EC)(dst)
        lv[n] = (src, dst, d0, d1)

    h = _cheb(x, *lv[N6], W1, b1)
    x1 = h[:N5]
    h = _cheb(x1, *lv[N5], W2, b2)
    x2 = h[:N4]
    h = _cheb(x2, *lv[N4], W3, b3)
    x3 = h[:N3]
    h = _cheb(x3, *lv[N3], W4, b4)
    x4 = h[:N2]
    h = _cheb(x4, *lv[N2], W5, b5)
    x5 = h[:N1]
    h = _cheb(x5, *lv[N1], W6, b6)

    h = jnp.concatenate([_upsample(h, up2), x4], axis=1)
    h = _cheb(h, *lv[N2], W7, b7)
    h = jnp.concatenate([_upsample(h, up3), x3], axis=1)
    h = _cheb(h, *lv[N3], W8, b8)
    h = jnp.concatenate([_upsample(h, up4), x2], axis=1)
    h = _cheb(h, *lv[N4], W9, b9)
    h = jnp.concatenate([_upsample(h, up5), x1], axis=1)
    h = _cheb(h, *lv[N5], W10, b10)
    h = jnp.concatenate([_upsample(h, up6), x], axis=1)
    return _cheb(h, *lv[N6], W11, b11, final=True)


# speedup vs baseline: 10.7111x; 1.7597x over previous
import functools

import jax
import jax.numpy as jnp
from jax import lax
from jax.experimental import pallas as pl
from jax.experimental.pallas import tpu as pltpu
from jax.experimental.pallas import tpu_sc as plsc

NC = 2
NS = 16
NW = NC * NS
EC = 128


def _rup(x, m):
    return -(-x // m) * m


def _zero_fill(ref, nrows, width):
    z = jnp.zeros((16,), jnp.float32)

    def body(i, _):
        for j in range(width // 16):
            ref[i, pl.ds(16 * j, 16)] = z
        if width % 16:
            ref[i, pl.ds(width - 16, 16)] = z
        return 0

    lax.fori_loop(0, nrows, body, 0)


def _copy_slabs(src_at, dst_at, tid, rpt, zr, nzc, nacc, sem):
    descs = []
    for z in range(nzc):
        start = jnp.minimum(tid * rpt + z * zr, nacc - zr)
        descs.append(pltpu.async_copy(src_at(start), dst_at(start), sem))
    for d in descs:
        d.wait()


def _acc_geom(N):
    Nacc = _rup(N + 1, 8)
    rpt = _rup(-(-Nacc // NS), 8)
    zr = min(rpt, 64)
    nzc = -(-rpt // zr)
    return Nacc, rpt, zr, nzc


@functools.cache
def _make_deg(N, Ep, ec):
    Nacc, rpt, zr, nzc = _acc_geom(N)
    n_chunks = Ep // ec // NW
    mesh = plsc.VectorSubcoreMesh(core_axis_name="c", subcore_axis_name="s")

    @functools.partial(
        pl.kernel,
        mesh=mesh,
        compiler_params=pltpu.CompilerParams(use_tc_tiling_on_sc=False),
        out_type=(
            jax.ShapeDtypeStruct((Nacc, 16), jnp.float32),
            jax.ShapeDtypeStruct((Nacc, 16), jnp.float32),
        ),
        name=f"sc_deg_n{N}",
        scratch_types=[
            pltpu.VMEM((min(n_chunks, 16), ec), jnp.int32),
            pltpu.VMEM((ec, 16), jnp.float32),
            pltpu.VMEM((zr, 16), jnp.float32),
            pltpu.VMEM_SHARED((Nacc, 16), jnp.float32),
            pltpu.SemaphoreType.DMA,
        ],
    )
    def deg_kernel(dst_hbm, out0, out1, dst_slab, ones_v, zrows_v, acc, csem):
        cid = lax.axis_index("c")
        tid = lax.axis_index("s")
        wid = tid * NC + cid

        one = jnp.ones((16,), jnp.float32)

        def fill_ones(i, _):
            ones_v[i, pl.ds(0, 16)] = one
            return 0

        lax.fori_loop(0, ec, fill_ones, 0)
        _zero_fill(zrows_v, zr, 16)
        _copy_slabs(
            lambda s: zrows_v.at[:],
            lambda s: acc.at[pl.ds(s, zr), :],
            tid, rpt, zr, nzc, Nacc, csem,
        )
        plsc.subcore_barrier()

        sb = min(n_chunks, 16)
        for s0 in range(0, n_chunks, sb):
            sbc = min(sb, n_chunks - s0)
            pltpu.sync_copy(
                dst_hbm.at[pl.ds(wid * n_chunks + s0, sbc), :],
                dst_slab.at[pl.ds(0, sbc)])
            for k2 in range(sbc):
                pltpu.sync_copy(ones_v, acc.at[dst_slab.at[k2]], add=True)
        plsc.subcore_barrier()

        @pl.when(cid == 0)
        def _():
            _copy_slabs(
                lambda s: acc.at[pl.ds(s, zr), :],
                lambda s: out0.at[pl.ds(s, zr), :],
                tid, rpt, zr, nzc, Nacc, csem,
            )

        @pl.when(cid == 1)
        def _():
            _copy_slabs(
                lambda s: acc.at[pl.ds(s, zr), :],
                lambda s: out1.at[pl.ds(s, zr), :],
                tid, rpt, zr, nzc, Nacc, csem,
            )

    return deg_kernel


@functools.cache
def _make_segsum(N, Fp, Ep, ec):
    Nacc, rpt, zr, nzc = _acc_geom(N)
    n_chunks = Ep // ec // NW
    db = min(3 if Fp <= 192 else 1, n_chunks)
    mesh = plsc.VectorSubcoreMesh(core_axis_name="c", subcore_axis_name="s")

    @functools.partial(
        pl.kernel,
        mesh=mesh,
        compiler_params=pltpu.CompilerParams(use_tc_tiling_on_sc=False),
        out_type=(
            jax.ShapeDtypeStruct((Nacc, Fp), jnp.float32),
            jax.ShapeDtypeStruct((Nacc, Fp), jnp.float32),
        ),
        name=f"sc_segsum_n{N}_f{Fp}",
        scratch_types=[
            pltpu.VMEM((min(n_chunks, 16), ec), jnp.int32),
            pltpu.VMEM((min(n_chunks, 16), ec), jnp.int32),
            [pltpu.VMEM((ec, Fp), jnp.float32) for _ in range(db)],
            [pltpu.SemaphoreType.DMA for _ in range(db)],
            pltpu.VMEM((zr, Fp), jnp.float32),
            pltpu.VMEM_SHARED((Nacc, Fp), jnp.float32),
            pltpu.SemaphoreType.DMA,
        ],
    )
    def segsum_kernel(g_hbm, src_hbm, dst_hbm, out0, out1,
                      src_slab, dst_slab, rows, sems, zrows_v, acc, csem):
        cid = lax.axis_index("c")
        tid = lax.axis_index("s")
        wid = tid * NC + cid

        _zero_fill(zrows_v, zr, Fp)
        _copy_slabs(
            lambda s: zrows_v.at[:],
            lambda s: acc.at[pl.ds(s, zr), :],
            tid, rpt, zr, nzc, Nacc, csem,
        )
        plsc.subcore_barrier()

        sb = min(n_chunks, 16)
        for s0 in range(0, n_chunks, sb):
            sbc = min(sb, n_chunks - s0)
            pltpu.sync_copy(
                src_hbm.at[pl.ds(wid * n_chunks + s0, sbc), :],
                src_slab.at[pl.ds(0, sbc)])
            pltpu.sync_copy(
                dst_hbm.at[pl.ds(wid * n_chunks + s0, sbc), :],
                dst_slab.at[pl.ds(0, sbc)])
            dbc = min(db, sbc)
            descs = [None] * dbc
            for k in range(dbc):
                descs[k] = pltpu.async_copy(
                    g_hbm.at[src_slab.at[k]], rows[k], sems[k])
            for k in range(sbc):
                b = k % dbc
                descs[b].wait()
                pltpu.sync_copy(rows[b], acc.at[dst_slab.at[k]], add=True)
                nk = k + dbc
                if nk < sbc:
                    descs[b] = pltpu.async_copy(
                        g_hbm.at[src_slab.at[nk]], rows[b], sems[b])
        plsc.subcore_barrier()

        @pl.when(cid == 0)
        def _():
            _copy_slabs(
                lambda s: acc.at[pl.ds(s, zr), :],
                lambda s: out0.at[pl.ds(s, zr), :],
                tid, rpt, zr, nzc, Nacc, csem,
            )

        @pl.when(cid == 1)
        def _():
            _copy_slabs(
                lambda s: acc.at[pl.ds(s, zr), :],
                lambda s: out1.at[pl.ds(s, zr), :],
                tid, rpt, zr, nzc, Nacc, csem,
            )

    return segsum_kernel


@functools.cache
def _make_upsample(Nprev, F, Rp, cu):
    r_per_w = Rp // NW
    n_chunks = r_per_w // cu
    mesh = plsc.VectorSubcoreMesh(core_axis_name="c", subcore_axis_name="s")

    @functools.partial(
        pl.kernel,
        mesh=mesh,
        compiler_params=pltpu.CompilerParams(use_tc_tiling_on_sc=False),
        name=f"sc_upsample_r{Rp}_f{F}",
        out_type=jax.ShapeDtypeStruct((Rp, F), jnp.float32),
        scratch_types=[
            pltpu.VMEM((n_chunks, cu), jnp.int32),
            pltpu.VMEM((n_chunks, cu), jnp.int32),
            pltpu.VMEM((cu, F), jnp.float32),
            pltpu.VMEM((cu, F), jnp.float32),
            pltpu.SemaphoreType.DMA,
            pltpu.SemaphoreType.DMA,
        ],
    )
    def up_kernel(h_hbm, i0_hbm, i1_hbm, out, i0_slab, i1_slab,
                  r0_v, r1_v, sem0, sem1):
        cid = lax.axis_index("c")
        tid = lax.axis_index("s")
        wid = tid * NC + cid

        pltpu.sync_copy(i0_hbm.at[pl.ds(wid * n_chunks, n_chunks), :], i0_slab)
        pltpu.sync_copy(i1_hbm.at[pl.ds(wid * n_chunks, n_chunks), :], i1_slab)
        for k in range(n_chunks):
            base = (wid * n_chunks + k) * cu
            d0 = pltpu.async_copy(h_hbm.at[i0_slab.at[k]], r0_v, sem0)
            d1 = pltpu.async_copy(h_hbm.at[i1_slab.at[k]], r1_v, sem1)
            d0.wait()
            d1.wait()

            def row(r, _):
                for j in range(F // 16):
                    s = pl.ds(16 * j, 16)
                    r0_v[r, s] = (r0_v[r, s] + r1_v[r, s]) * 0.5
                return 0

            lax.fori_loop(0, cu, row, 0)
            pltpu.sync_copy(r0_v, out.at[pl.ds(base, cu), :])

    return up_kernel


def _dinv_block(d0_ref, d1_ref):
    deg = d0_ref[:, :1] + d1_ref[:, :1]
    return jnp.where(deg > 0, lax.rsqrt(jnp.maximum(deg, 1.0)), 0.0)


@functools.cache
def _make_scale1(N, Fi, Fp, bn):
    def body(h_ref, d0_ref, d1_ref, o_ref):
        dinv = _dinv_block(d0_ref, d1_ref)
        o_ref[:, :Fi] = h_ref[...] * dinv
        if Fp > Fi:
            o_ref[:, Fi:] = jnp.zeros((bn, Fp - Fi), jnp.float32)

    grid = -(-N // bn)
    return pl.pallas_call(
        body,
        grid=(grid,),
        in_specs=[
            pl.BlockSpec((bn, Fi), lambda i: (i, 0)),
            pl.BlockSpec((bn, 16), lambda i: (i, 0)),
            pl.BlockSpec((bn, 16), lambda i: (i, 0)),
        ],
        out_specs=pl.BlockSpec((bn, Fp), lambda i: (i, 0)),
        out_shape=jax.ShapeDtypeStruct((N, Fp), jnp.float32),
    )


@functools.cache
def _make_scale2(N, Fp, bn):
    def body(s1a_ref, s1b_ref, d0_ref, d1_ref, o_ref):
        deg = d0_ref[:, :1] + d1_ref[:, :1]
        idinv2 = jnp.where(deg > 0, -1.0 / jnp.maximum(deg, 1.0), 0.0)
        o_ref[...] = (s1a_ref[...] + s1b_ref[...]) * idinv2

    grid = -(-N // bn)
    return pl.pallas_call(
        body,
        grid=(grid,),
        in_specs=[
            pl.BlockSpec((bn, Fp), lambda i: (i, 0)),
            pl.BlockSpec((bn, Fp), lambda i: (i, 0)),
            pl.BlockSpec((bn, 16), lambda i: (i, 0)),
            pl.BlockSpec((bn, 16), lambda i: (i, 0)),
        ],
        out_specs=pl.BlockSpec((bn, Fp), lambda i: (i, 0)),
        out_shape=jax.ShapeDtypeStruct((N, Fp), jnp.float32),
    )


@functools.cache
def _make_combine(N, Fi, Fp, Fo, bn, final):
    def body(h_ref, s1a_ref, s1b_ref, s2a_ref, s2b_ref, d0_ref, d1_ref,
             w_ref, b_ref, o_ref):
        dinv = _dinv_block(d0_ref, d1_ref)
        s1 = (s1a_ref[:, :Fi] + s1b_ref[:, :Fi]) * dinv
        s2 = (s2a_ref[:, :Fi] + s2b_ref[:, :Fi]) * dinv
        w0 = w_ref[0]
        w1 = w_ref[1]
        w2 = w_ref[2]
        acc = jnp.dot(h_ref[...], w0 - w2, preferred_element_type=jnp.float32)
        acc = acc - jnp.dot(s1, w1, preferred_element_type=jnp.float32)
        acc = acc - 2.0 * jnp.dot(s2, w2, preferred_element_type=jnp.float32)
        acc = acc + b_ref[...]
        if final:
            m = jnp.max(acc, axis=1, keepdims=True)
            e = jnp.exp(acc - m)
            o_ref[...] = e / jnp.sum(e, axis=1, keepdims=True)
        else:
            o_ref[...] = jnp.maximum(acc, 0.0)

    grid = -(-N // bn)
    return pl.pallas_call(
        body,
        grid=(grid,),
        in_specs=[
            pl.BlockSpec((bn, Fi), lambda i: (i, 0)),
            pl.BlockSpec((bn, Fp), lambda i: (i, 0)),
            pl.BlockSpec((bn, Fp), lambda i: (i, 0)),
            pl.BlockSpec((bn, Fp), lambda i: (i, 0)),
            pl.BlockSpec((bn, Fp), lambda i: (i, 0)),
            pl.BlockSpec((bn, 16), lambda i: (i, 0)),
            pl.BlockSpec((bn, 16), lambda i: (i, 0)),
            pl.BlockSpec((3, Fi, Fo), lambda i: (0, 0, 0)),
            pl.BlockSpec((1, Fo), lambda i: (0, 0)),
        ],
        out_specs=pl.BlockSpec((bn, Fo), lambda i: (i, 0)),
        out_shape=jax.ShapeDtypeStruct((N, Fo), jnp.float32),
    )


def _prep_edges(edge_index, N):
    E = edge_index.shape[1]
    Ep = _rup(E, NW * 8)
    e_per_w = Ep // NW
    ec = max(d for d in range(8, 129, 8) if e_per_w % d == 0)
    src = jnp.pad(edge_index[0], (0, Ep - E)).reshape(Ep // ec, ec)
    dst = jnp.pad(edge_index[1], (0, Ep - E),
                  constant_values=N).reshape(Ep // ec, ec)
    return src, dst, ec


def _cheb(h, src, dst, ec_unused, deg0, deg1, W, b, final=False):
    N, Fi = h.shape
    Fp = _rup(Fi, 16)
    if _rup(N + 1, 8) * Fp > 1_800_000:
        Fp = _rup(Fi, 8)
    Fo = W.shape[2]
    ec = src.shape[1]
    Ep = src.shape[0] * ec
    bn = min(1024, _rup(N, 8))
    g0 = _make_scale1(N, Fi, Fp, bn)(h, deg0, deg1)
    s1a, s1b = _make_segsum(N, Fp, Ep, ec)(g0, src, dst)
    g1 = _make_scale2(N, Fp, bn)(s1a, s1b, deg0, deg1)
    s2a, s2b = _make_segsum(N, Fp, Ep, ec)(g1, src, dst)
    return _make_combine(N, Fi, Fp, Fo, bn, final)(
        h, s1a, s1b, s2a, s2b, deg0, deg1, W, b.reshape(1, -1))


def _upsample(h, up):
    Nprev, F = h.shape
    R = up.shape[0]
    r_per_w = _rup(-(-R // NW), 8)
    cu = max(d for d in range(8, 129, 8) if r_per_w % d == 0)
    Rp = NW * r_per_w
    i0 = jnp.pad(up[:, 0], (0, Rp - R)).reshape(Rp // cu, cu)
    i1 = jnp.pad(up[:, 1], (0, Rp - R)).reshape(Rp // cu, cu)
    new = _make_upsample(Nprev, F, Rp, cu)(h, i0, i1)
    return jnp.concatenate([h, new[:R]], axis=0)


def kernel(x, edge_index, edge_index_1, edge_index_2, edge_index_3,
           edge_index_4, edge_index_5, up2, up3, up4, up5, up6,
           W1, b1, W2, b2, W3, b3, W4, b4, W5, b5, W6, b6, W7, b7,
           W8, b8, W9, b9, W10, b10, W11, b11):
    N6, N5, N4, N3, N2, N1 = 40962, 10242, 2562, 642, 162, 42
    lv = {}
    for n, ei in ((N6, edge_index), (N5, edge_index_5), (N4, edge_index_4),
                  (N3, edge_index_3), (N2, edge_index_2), (N1, edge_index_1)):
        src, dst, ec = _prep_edges(ei, n)
        d0, d1 = _make_deg(n, src.shape[0] * ec, ec)(dst)
        lv[n] = (src, dst, ec, d0, d1)

    h = _cheb(x, *lv[N6], W1, b1)
    x1 = h[:N5]
    h = _cheb(x1, *lv[N5], W2, b2)
    x2 = h[:N4]
    h = _cheb(x2, *lv[N4], W3, b3)
    x3 = h[:N3]
    h = _cheb(x3, *lv[N3], W4, b4)
    x4 = h[:N2]
    h = _cheb(x4, *lv[N2], W5, b5)
    x5 = h[:N1]
    h = _cheb(x5, *lv[N1], W6, b6)

    h = jnp.concatenate([_upsample(h, up2), x4], axis=1)
    h = _cheb(h, *lv[N2], W7, b7)
    h = jnp.concatenate([_upsample(h, up3), x3], axis=1)
    h = _cheb(h, *lv[N3], W8, b8)
    h = jnp.concatenate([_upsample(h, up4), x2], axis=1)
    h = _cheb(h, *lv[N4], W9, b9)
    h = jnp.concatenate([_upsample(h, up5), x1], axis=1)
    h = _cheb(h, *lv[N5], W10, b10)
    h = jnp.concatenate([_upsample(h, up6), x], axis=1)
    return _cheb(h, *lv[N6], W11, b11, final=True)


# speedup vs baseline: 10.7523x; 1.0038x over previous
import functools

import jax
import jax.numpy as jnp
from jax import lax
from jax.experimental import pallas as pl
from jax.experimental.pallas import tpu as pltpu
from jax.experimental.pallas import tpu_sc as plsc

NC = 2
NS = 16
NW = NC * NS
EC = 128


def _rup(x, m):
    return -(-x // m) * m


def _zero_fill(ref, nrows, width):
    z = jnp.zeros((16,), jnp.float32)

    def body(i, _):
        for j in range(width // 16):
            ref[i, pl.ds(16 * j, 16)] = z
        if width % 16:
            ref[i, pl.ds(width - 16, 16)] = z
        return 0

    lax.fori_loop(0, nrows, body, 0)


def _fire_slabs(src_at, dst_at, tid, rpt, zr, nzc, nacc, sem):
    descs = []
    for z in range(nzc):
        start = jnp.minimum(tid * rpt + z * zr, nacc - zr)
        descs.append(pltpu.async_copy(src_at(start), dst_at(start), sem))
    return descs


def _copy_slabs(src_at, dst_at, tid, rpt, zr, nzc, nacc, sem):
    for d in _fire_slabs(src_at, dst_at, tid, rpt, zr, nzc, nacc, sem):
        d.wait()


def _acc_geom(N):
    Nacc = _rup(N + 1, 8)
    rpt = _rup(-(-Nacc // NS), 8)
    zr = min(rpt, 64)
    nzc = -(-rpt // zr)
    return Nacc, rpt, zr, nzc


@functools.cache
def _make_deg(N, Ep, ec):
    Nacc, rpt, zr, nzc = _acc_geom(N)
    n_chunks = Ep // ec // NW
    mesh = plsc.VectorSubcoreMesh(core_axis_name="c", subcore_axis_name="s")

    @functools.partial(
        pl.kernel,
        mesh=mesh,
        compiler_params=pltpu.CompilerParams(use_tc_tiling_on_sc=False),
        out_type=(
            jax.ShapeDtypeStruct((Nacc, 16), jnp.float32),
            jax.ShapeDtypeStruct((Nacc, 16), jnp.float32),
        ),
        name=f"sc_deg_n{N}",
        scratch_types=[
            pltpu.VMEM((min(n_chunks, 16), ec), jnp.int32),
            pltpu.VMEM((ec, 16), jnp.float32),
            pltpu.VMEM((zr, 16), jnp.float32),
            pltpu.VMEM_SHARED((Nacc, 16), jnp.float32),
            pltpu.SemaphoreType.DMA,
        ],
    )
    def deg_kernel(dst_hbm, out0, out1, dst_slab, ones_v, zrows_v, acc, csem):
        cid = lax.axis_index("c")
        tid = lax.axis_index("s")
        wid = tid * NC + cid

        one = jnp.ones((16,), jnp.float32)

        def fill_ones(i, _):
            ones_v[i, pl.ds(0, 16)] = one
            return 0

        lax.fori_loop(0, ec, fill_ones, 0)
        _zero_fill(zrows_v, zr, 16)
        _copy_slabs(
            lambda s: zrows_v.at[:],
            lambda s: acc.at[pl.ds(s, zr), :],
            tid, rpt, zr, nzc, Nacc, csem,
        )
        plsc.subcore_barrier()

        sb = min(n_chunks, 16)
        for s0 in range(0, n_chunks, sb):
            sbc = min(sb, n_chunks - s0)
            pltpu.sync_copy(
                dst_hbm.at[pl.ds(wid * n_chunks + s0, sbc), :],
                dst_slab.at[pl.ds(0, sbc)])
            for k2 in range(sbc):
                pltpu.sync_copy(ones_v, acc.at[dst_slab.at[k2]], add=True)
        plsc.subcore_barrier()

        @pl.when(cid == 0)
        def _():
            _copy_slabs(
                lambda s: acc.at[pl.ds(s, zr), :],
                lambda s: out0.at[pl.ds(s, zr), :],
                tid, rpt, zr, nzc, Nacc, csem,
            )

        @pl.when(cid == 1)
        def _():
            _copy_slabs(
                lambda s: acc.at[pl.ds(s, zr), :],
                lambda s: out1.at[pl.ds(s, zr), :],
                tid, rpt, zr, nzc, Nacc, csem,
            )

    return deg_kernel


@functools.cache
def _make_segsum(N, Fp, Ep, ec):
    Nacc, rpt, zr, nzc = _acc_geom(N)
    n_chunks = Ep // ec // NW
    db = min(3 if Fp <= 192 else 1, n_chunks)
    mesh = plsc.VectorSubcoreMesh(core_axis_name="c", subcore_axis_name="s")

    @functools.partial(
        pl.kernel,
        mesh=mesh,
        compiler_params=pltpu.CompilerParams(use_tc_tiling_on_sc=False),
        out_type=(
            jax.ShapeDtypeStruct((Nacc, Fp), jnp.float32),
            jax.ShapeDtypeStruct((Nacc, Fp), jnp.float32),
        ),
        name=f"sc_segsum_n{N}_f{Fp}",
        scratch_types=[
            pltpu.VMEM((min(n_chunks, 16), ec), jnp.int32),
            pltpu.VMEM((min(n_chunks, 16), ec), jnp.int32),
            [pltpu.VMEM((ec, Fp), jnp.float32) for _ in range(db)],
            [pltpu.SemaphoreType.DMA for _ in range(db)],
            pltpu.VMEM((zr, Fp), jnp.float32),
            pltpu.VMEM_SHARED((Nacc, Fp), jnp.float32),
            pltpu.SemaphoreType.DMA,
        ],
    )
    def segsum_kernel(g_hbm, src_hbm, dst_hbm, out0, out1,
                      src_slab, dst_slab, rows, sems, zrows_v, acc, csem):
        cid = lax.axis_index("c")
        tid = lax.axis_index("s")
        wid = tid * NC + cid

        _zero_fill(zrows_v, zr, Fp)
        _copy_slabs(
            lambda s: zrows_v.at[:],
            lambda s: acc.at[pl.ds(s, zr), :],
            tid, rpt, zr, nzc, Nacc, csem,
        )
        plsc.subcore_barrier()

        sb = min(n_chunks, 16)
        for s0 in range(0, n_chunks, sb):
            sbc = min(sb, n_chunks - s0)
            pltpu.sync_copy(
                src_hbm.at[pl.ds(wid * n_chunks + s0, sbc), :],
                src_slab.at[pl.ds(0, sbc)])
            pltpu.sync_copy(
                dst_hbm.at[pl.ds(wid * n_chunks + s0, sbc), :],
                dst_slab.at[pl.ds(0, sbc)])
            dbc = min(db, sbc)
            descs = [None] * dbc
            for k in range(dbc):
                descs[k] = pltpu.async_copy(
                    g_hbm.at[src_slab.at[k]], rows[k], sems[k])
            for k in range(sbc):
                b = k % dbc
                descs[b].wait()
                pltpu.sync_copy(rows[b], acc.at[dst_slab.at[k]], add=True)
                nk = k + dbc
                if nk < sbc:
                    descs[b] = pltpu.async_copy(
                        g_hbm.at[src_slab.at[nk]], rows[b], sems[b])
        plsc.subcore_barrier()

        @pl.when(cid == 0)
        def _():
            _copy_slabs(
                lambda s: acc.at[pl.ds(s, zr), :],
                lambda s: out0.at[pl.ds(s, zr), :],
                tid, rpt, zr, nzc, Nacc, csem,
            )

        @pl.when(cid == 1)
        def _():
            _copy_slabs(
                lambda s: acc.at[pl.ds(s, zr), :],
                lambda s: out1.at[pl.ds(s, zr), :],
                tid, rpt, zr, nzc, Nacc, csem,
            )

    return segsum_kernel


@functools.cache
def _make_upsample(Nprev, F, Rp, cu):
    r_per_w = Rp // NW
    n_chunks = r_per_w // cu
    mesh = plsc.VectorSubcoreMesh(core_axis_name="c", subcore_axis_name="s")

    @functools.partial(
        pl.kernel,
        mesh=mesh,
        compiler_params=pltpu.CompilerParams(use_tc_tiling_on_sc=False),
        name=f"sc_upsample_r{Rp}_f{F}",
        out_type=jax.ShapeDtypeStruct((Rp, F), jnp.float32),
        scratch_types=[
            pltpu.VMEM((n_chunks, cu), jnp.int32),
            pltpu.VMEM((n_chunks, cu), jnp.int32),
            pltpu.VMEM((cu, F), jnp.float32),
            pltpu.VMEM((cu, F), jnp.float32),
            pltpu.SemaphoreType.DMA,
            pltpu.SemaphoreType.DMA,
        ],
    )
    def up_kernel(h_hbm, i0_hbm, i1_hbm, out, i0_slab, i1_slab,
                  r0_v, r1_v, sem0, sem1):
        cid = lax.axis_index("c")
        tid = lax.axis_index("s")
        wid = tid * NC + cid

        pltpu.sync_copy(i0_hbm.at[pl.ds(wid * n_chunks, n_chunks), :], i0_slab)
        pltpu.sync_copy(i1_hbm.at[pl.ds(wid * n_chunks, n_chunks), :], i1_slab)
        for k in range(n_chunks):
            base = (wid * n_chunks + k) * cu
            d0 = pltpu.async_copy(h_hbm.at[i0_slab.at[k]], r0_v, sem0)
            d1 = pltpu.async_copy(h_hbm.at[i1_slab.at[k]], r1_v, sem1)
            d0.wait()
            d1.wait()

            def row(r, _):
                for j in range(F // 16):
                    s = pl.ds(16 * j, 16)
                    r0_v[r, s] = (r0_v[r, s] + r1_v[r, s]) * 0.5
                return 0

            lax.fori_loop(0, cu, row, 0)
            pltpu.sync_copy(r0_v, out.at[pl.ds(base, cu), :])

    return up_kernel


def _dinv_block(d0_ref, d1_ref):
    deg = d0_ref[:, :1] + d1_ref[:, :1]
    return jnp.where(deg > 0, lax.rsqrt(jnp.maximum(deg, 1.0)), 0.0)


@functools.cache
def _make_scale1(N, Fi, Fp, bn):
    def body(h_ref, d0_ref, d1_ref, o_ref):
        dinv = _dinv_block(d0_ref, d1_ref)
        o_ref[:, :Fi] = h_ref[...] * dinv
        if Fp > Fi:
            o_ref[:, Fi:] = jnp.zeros((bn, Fp - Fi), jnp.float32)

    grid = -(-N // bn)
    return pl.pallas_call(
        body,
        grid=(grid,),
        in_specs=[
            pl.BlockSpec((bn, Fi), lambda i: (i, 0)),
            pl.BlockSpec((bn, 16), lambda i: (i, 0)),
            pl.BlockSpec((bn, 16), lambda i: (i, 0)),
        ],
        out_specs=pl.BlockSpec((bn, Fp), lambda i: (i, 0)),
        out_shape=jax.ShapeDtypeStruct((N, Fp), jnp.float32),
    )


@functools.cache
def _make_scale2(N, Fp, bn):
    def body(s1a_ref, s1b_ref, d0_ref, d1_ref, o_ref):
        deg = d0_ref[:, :1] + d1_ref[:, :1]
        idinv2 = jnp.where(deg > 0, -1.0 / jnp.maximum(deg, 1.0), 0.0)
        o_ref[...] = (s1a_ref[...] + s1b_ref[...]) * idinv2

    grid = -(-N // bn)
    return pl.pallas_call(
        body,
        grid=(grid,),
        in_specs=[
            pl.BlockSpec((bn, Fp), lambda i: (i, 0)),
            pl.BlockSpec((bn, Fp), lambda i: (i, 0)),
            pl.BlockSpec((bn, 16), lambda i: (i, 0)),
            pl.BlockSpec((bn, 16), lambda i: (i, 0)),
        ],
        out_specs=pl.BlockSpec((bn, Fp), lambda i: (i, 0)),
        out_shape=jax.ShapeDtypeStruct((N, Fp), jnp.float32),
    )


@functools.cache
def _make_combine(N, Fi, Fp, Fo, bn, final, nxt=None, nxt_rows=0):
    def body(*refs):
        if nxt:
            (h_ref, s1a_ref, s1b_ref, s2a_ref, s2b_ref, d0_ref, d1_ref,
             w_ref, b_ref, dn0_ref, dn1_ref, o_ref, o2_ref) = refs
        else:
            (h_ref, s1a_ref, s1b_ref, s2a_ref, s2b_ref, d0_ref, d1_ref,
             w_ref, b_ref, o_ref) = refs
        dinv = _dinv_block(d0_ref, d1_ref)
        s1 = (s1a_ref[:, :Fi] + s1b_ref[:, :Fi]) * dinv
        s2 = (s2a_ref[:, :Fi] + s2b_ref[:, :Fi]) * dinv
        w0 = w_ref[0]
        w1 = w_ref[1]
        w2 = w_ref[2]
        acc = jnp.dot(h_ref[...], w0 - w2, preferred_element_type=jnp.float32)
        acc = acc - jnp.dot(s1, w1, preferred_element_type=jnp.float32)
        acc = acc - 2.0 * jnp.dot(s2, w2, preferred_element_type=jnp.float32)
        acc = acc + b_ref[...]
        if final:
            m = jnp.max(acc, axis=1, keepdims=True)
            e = jnp.exp(acc - m)
            res = e / jnp.sum(e, axis=1, keepdims=True)
        else:
            res = jnp.maximum(acc, 0.0)
        o_ref[...] = res
        if nxt:
            (Fpn,) = nxt
            dinvn = _dinv_block(dn0_ref, dn1_ref)
            o2_ref[:, :Fo] = res * dinvn
            if Fpn > Fo:
                o2_ref[:, Fo:] = jnp.zeros((bn, Fpn - Fo), jnp.float32)

    grid = -(-N // bn)
    in_specs = [
        pl.BlockSpec((bn, Fi), lambda i: (i, 0)),
        pl.BlockSpec((bn, Fp), lambda i: (i, 0)),
        pl.BlockSpec((bn, Fp), lambda i: (i, 0)),
        pl.BlockSpec((bn, Fp), lambda i: (i, 0)),
        pl.BlockSpec((bn, Fp), lambda i: (i, 0)),
        pl.BlockSpec((bn, 16), lambda i: (i, 0)),
        pl.BlockSpec((bn, 16), lambda i: (i, 0)),
        pl.BlockSpec((3, Fi, Fo), lambda i: (0, 0, 0)),
        pl.BlockSpec((1, Fo), lambda i: (0, 0)),
    ]
    out_specs = pl.BlockSpec((bn, Fo), lambda i: (i, 0))
    out_shape = jax.ShapeDtypeStruct((N, Fo), jnp.float32)
    if nxt:
        (Fpn,) = nxt
        nb = max(0, (nxt_rows - 1) // bn)
        in_specs += [
            pl.BlockSpec((bn, 16), lambda i: (jnp.minimum(i, nb), 0)),
            pl.BlockSpec((bn, 16), lambda i: (jnp.minimum(i, nb), 0)),
        ]
        out_specs = [out_specs, pl.BlockSpec((bn, Fpn), lambda i: (i, 0))]
        out_shape = [out_shape,
                     jax.ShapeDtypeStruct((N, Fpn), jnp.float32)]
    return pl.pallas_call(
        body,
        grid=(grid,),
        in_specs=in_specs,
        out_specs=out_specs,
        out_shape=out_shape,
    )


def _prep_edges(edge_index, N):
    E = edge_index.shape[1]
    Ep = _rup(E, NW * 8)
    e_per_w = Ep // NW
    ec = max(d for d in range(8, 129, 8) if e_per_w % d == 0)
    src = jnp.pad(edge_index[0], (0, Ep - E)).reshape(Ep // ec, ec)
    dst = jnp.pad(edge_index[1], (0, Ep - E),
                  constant_values=N).reshape(Ep // ec, ec)
    return src, dst, ec


def _fp_of(N, Fi):
    Fp = _rup(Fi, 16)
    if _rup(N + 1, 8) * Fp > 1_800_000:
        Fp = _rup(Fi, 8)
    return Fp


def _cheb(h, lvl, W, b, final=False, g0=None, nxt=None):
    src, dst, ec, deg0, deg1 = lvl
    N, Fi = h.shape
    Fp = _fp_of(N, Fi)
    Fo = W.shape[2]
    Ep = src.shape[0] * ec
    bn = min(1024, _rup(N, 8))
    if g0 is None:
        g0 = _make_scale1(N, Fi, Fp, bn)(h, deg0, deg1)
    s1a, s1b = _make_segsum(N, Fp, Ep, ec)(g0, src, dst)
    g1 = _make_scale2(N, Fp, bn)(s1a, s1b, deg0, deg1)
    s2a, s2b = _make_segsum(N, Fp, Ep, ec)(g1, src, dst)
    cargs = [h, s1a, s1b, s2a, s2b, deg0, deg1, W, b.reshape(1, -1)]
    cn = None
    nxt_rows = 0
    if nxt is not None:
        dn0, dn1 = nxt
        cn = (_fp_of(dn0.shape[0], Fo),)
        nxt_rows = dn0.shape[0]
        cargs += [dn0, dn1]
    res = _make_combine(N, Fi, Fp, Fo, bn, final, cn, nxt_rows)(*cargs)
    if nxt is None:
        return res, None
    return res[0], res[1]


def _upsample(h, up):
    Nprev, F = h.shape
    R = up.shape[0]
    r_per_w = _rup(-(-R // NW), 8)
    cu = max(d for d in range(8, 129, 8) if r_per_w % d == 0)
    Rp = NW * r_per_w
    i0 = jnp.pad(up[:, 0], (0, Rp - R)).reshape(Rp // cu, cu)
    i1 = jnp.pad(up[:, 1], (0, Rp - R)).reshape(Rp // cu, cu)
    new = _make_upsample(Nprev, F, Rp, cu)(h, i0, i1)
    return jnp.concatenate([h, new[:R]], axis=0)


def kernel(x, edge_index, edge_index_1, edge_index_2, edge_index_3,
           edge_index_4, edge_index_5, up2, up3, up4, up5, up6,
           W1, b1, W2, b2, W3, b3, W4, b4, W5, b5, W6, b6, W7, b7,
           W8, b8, W9, b9, W10, b10, W11, b11):
    N6, N5, N4, N3, N2, N1 = 40962, 10242, 2562, 642, 162, 42
    ns = (N6, N5, N4, N3, N2, N1)
    eis = (edge_index, edge_index_5, edge_index_4, edge_index_3,
           edge_index_2, edge_index_1)
    lv = {}
    for n, ei in zip(ns, eis):
        src, dst, ec = _prep_edges(ei, n)
        d0, d1 = _make_deg(n, src.shape[0] * ec, ec)(dst)
        lv[n] = (src, dst, ec, d0, d1)

    def dn(n):
        return lv[n][3], lv[n][4]

    h, g0n = _cheb(x, lv[N6], W1, b1, nxt=dn(N5))
    x1 = h[:N5]
    h, g0n = _cheb(x1, lv[N5], W2, b2, g0=g0n, nxt=dn(N4))
    x2 = h[:N4]
    h, g0n = _cheb(x2, lv[N4], W3, b3, g0=g0n, nxt=dn(N3))
    x3 = h[:N3]
    h, g0n = _cheb(x3, lv[N3], W4, b4, g0=g0n, nxt=dn(N2))
    x4 = h[:N2]
    h, g0n = _cheb(x4, lv[N2], W5, b5, g0=g0n, nxt=dn(N1))
    x5 = h[:N1]
    h, _ = _cheb(x5, lv[N1], W6, b6, g0=g0n)

    h = jnp.concatenate([_upsample(h, up2), x4], axis=1)
    h, _ = _cheb(h, lv[N2], W7, b7)
    h = jnp.concatenate([_upsample(h, up3), x3], axis=1)
    h, _ = _cheb(h, lv[N3], W8, b8)
    h = jnp.concatenate([_upsample(h, up4), x2], axis=1)
    h, _ = _cheb(h, lv[N4], W9, b9)
    h = jnp.concatenate([_upsample(h, up5), x1], axis=1)
    h, _ = _cheb(h, lv[N5], W10, b10)
    h = jnp.concatenate([_upsample(h, up6), x], axis=1)
    out, _ = _cheb(h, lv[N6], W11, b11, final=True)
    return out
